# Initial kernel scaffold; baseline (speedup 1.0000x reference)
import jax
import jax.numpy as jnp
from jax.experimental import pallas as pl


def kernel(x, logits, W1, b1, W2, b2, W3, b3, epoch, total_epochs, training):
    raise NotImplementedError("write your pallas kernel here")


# speedup vs baseline: 1.5858x; 1.5858x over previous
import functools

import jax
import jax.numpy as jnp
from jax.experimental import pallas as pl
from jax.experimental.pallas import tpu as pltpu

IN_DIM = 32768
OUT_DIM = 10
K = 1024
BATCH = 128
BLK = 2048
N_BLK = IN_DIM // BLK
INT_MIN = -2147483648


def _mask_body(l_ref, m_ref):
    l = l_ref[...]
    bits = jax.lax.bitcast_convert_type(l, jnp.int32)
    key = bits ^ jax.lax.shift_right_arithmetic(bits, 31) & jnp.int32(0x7FFFFFFF)
    key = jnp.where(bits == jnp.int32(INT_MIN), jnp.int32(0), key)

    def step(i, t_u):
        b = 31 - i
        cand_u = t_u | (jnp.int32(1) << b)
        cs = cand_u ^ jnp.int32(INT_MIN)
        cnt = jnp.sum((key >= cs).astype(jnp.int32))
        return jnp.where(cnt >= K, cand_u, t_u)

    t_u = jax.lax.fori_loop(0, 32, step, jnp.int32(0))
    thr = t_u ^ jnp.int32(INT_MIN)

    gt = key > thr
    eq = key == thr
    r = K - jnp.sum(gt.astype(jnp.int32))
    csum = eq.astype(jnp.int32)
    s = 1
    while s < IN_DIM:
        shifted = jnp.concatenate(
            [jnp.zeros((1, s), jnp.int32), csum[:, :-s]], axis=1)
        csum = csum + shifted
        s *= 2
    sel = eq & (csum <= r)
    m_ref[...] = (gt | sel).astype(jnp.float32)


def _mlp_body(x_ref, m_ref, w1_ref, b1_ref, w2_ref, b2_ref, w3_ref, b3_ref,
              out_ref, acc_ref):
    i = pl.program_id(0)

    @pl.when(i == 0)
    def _():
        acc_ref[...] = jnp.zeros_like(acc_ref)

    xm = x_ref[...] * m_ref[...]
    acc_ref[...] += jnp.dot(xm, w1_ref[...], preferred_element_type=jnp.float32)

    @pl.when(i == N_BLK - 1)
    def _():
        h = jnp.maximum(acc_ref[...] + b1_ref[...], 0.0)
        h = jnp.maximum(
            jnp.dot(h, w2_ref[...], preferred_element_type=jnp.float32)
            + b2_ref[...], 0.0)
        out_ref[...] = (
            jnp.dot(h, w3_ref[...], preferred_element_type=jnp.float32)
            + b3_ref[...])


@functools.partial(jax.jit, static_argnames=("interpret",))
def kernel(x, logits, W1, b1, W2, b2, W3, b3, epoch, total_epochs, training,
           interpret=False):
    del epoch, total_epochs, training
    l2 = logits.reshape(1, IN_DIM)
    mask2 = pl.pallas_call(
        _mask_body,
        out_shape=jax.ShapeDtypeStruct((1, IN_DIM), jnp.float32),
        interpret=interpret,
    )(l2)

    out = pl.pallas_call(
        _mlp_body,
        grid=(N_BLK,),
        in_specs=[
            pl.BlockSpec((BATCH, BLK), lambda i: (0, i)),
            pl.BlockSpec((1, BLK), lambda i: (0, i)),
            pl.BlockSpec((BLK, 32), lambda i: (i, 0)),
            pl.BlockSpec((1, 32), lambda i: (0, 0)),
            pl.BlockSpec((32, 16), lambda i: (0, 0)),
            pl.BlockSpec((1, 16), lambda i: (0, 0)),
            pl.BlockSpec((16, OUT_DIM), lambda i: (0, 0)),
            pl.BlockSpec((1, OUT_DIM), lambda i: (0, 0)),
        ],
        out_specs=pl.BlockSpec((BATCH, OUT_DIM), lambda i: (0, 0)),
        out_shape=jax.ShapeDtypeStruct((BATCH, OUT_DIM), jnp.float32),
        scratch_shapes=[pltpu.VMEM((BATCH, 32), jnp.float32)],
        interpret=interpret,
    )(x, mask2, W1, b1.reshape(1, 32), W2, b2.reshape(1, 16), W3,
      b3.reshape(1, OUT_DIM))

    return out, mask2.reshape(IN_DIM)
